# baseline (speedup 1.0000x reference)
import jax
import jax.numpy as jnp
from jax import lax
from jax.experimental import pallas as pl
from jax.experimental.pallas import tpu as pltpu
from jax.experimental.pallas import tpu_sc as plsc

N_NODES = 10000
D = 128
N_CORES = 2
N_SUB = 16
CHUNK = 128
SPLIT0 = 0.505
ROWS_PER_TILE = 624
TAIL_BASE = ROWS_PER_TILE * N_SUB
TAIL_ROWS = N_NODES - TAIL_BASE
ACC_ROWS = N_NODES


def _sc_agg(feats, ei, zeros):
    n_edges = ei.shape[1]
    t_chunks = n_edges // CHUNK
    c0 = int(round(t_chunks * SPLIT0))
    c1 = t_chunks - c0
    k0max = -(-c0 // N_SUB)
    k1max = -(-c1 // N_SUB)

    def body(feats, ei, zeros, out, idx_s, idx_d0, idx_d1, rows0, rows1, acc,
             gsem0, gsem1, ssem0, ssem1, dsem0, dsem1, stg):
        c = lax.axis_index("c")
        s = lax.axis_index("s")

        def cdims(cbase, csize):
            r_lo = s * csize // N_SUB
            cnt = (s + 1) * csize // N_SUB - r_lo
            off = pl.multiple_of((cbase + r_lo) * CHUNK, CHUNK)
            return off, cnt

        def stage(cbase, csize, kmax):
            off, _ = cdims(cbase, csize)
            pltpu.async_copy(ei.at[0, pl.ds(off, kmax * CHUNK)],
                             idx_s.at[pl.ds(0, kmax * CHUNK)], stg)

        @pl.when(c == 0)
        def _():
            stage(0, c0, k0max)

        @pl.when(c != 0)
        def _():
            stage(c0, c1, k1max)

        base = s * ROWS_PER_TILE

        @pl.when(c == 0)
        def _():
            pltpu.sync_copy(feats.at[pl.ds(base, ROWS_PER_TILE)],
                            acc.at[pl.ds(base, ROWS_PER_TILE)])

            @pl.when(s == N_SUB - 1)
            def _():
                pltpu.sync_copy(feats.at[pl.ds(TAIL_BASE, TAIL_ROWS)],
                                acc.at[pl.ds(TAIL_BASE, TAIL_ROWS)])

        @pl.when(c != 0)
        def _():
            for j in range(ROWS_PER_TILE // CHUNK):
                pltpu.sync_copy(zeros,
                                acc.at[pl.ds(base + j * CHUNK, CHUNK)])
            rem = ROWS_PER_TILE % CHUNK
            pltpu.sync_copy(zeros.at[pl.ds(0, rem)],
                            acc.at[pl.ds(base + ROWS_PER_TILE - rem, rem)])

            @pl.when(s == N_SUB - 1)
            def _():
                pltpu.sync_copy(zeros.at[pl.ds(0, TAIL_ROWS)],
                                acc.at[pl.ds(TAIL_BASE, TAIL_ROWS)])

        plsc.subcore_barrier()

        rows = (rows0, rows1)
        idx_d = (idx_d0, idx_d1)
        gsem = (gsem0, gsem1)
        ssem = (ssem0, ssem1)
        dsem = (dsem0, dsem1)

        def run(cbase, csize, kmax):
            off, cnt = cdims(cbase, csize)
            pltpu.make_async_copy(ei.at[0, pl.ds(0, kmax * CHUNK)],
                                  idx_s.at[pl.ds(0, kmax * CHUNK)], stg).wait()

            def issue_gather(k, b):
                kof = pl.multiple_of(k * CHUNK, CHUNK)
                pltpu.async_copy(feats.at[idx_s.at[pl.ds(kof, CHUNK)]],
                                 rows[b], gsem[b])
                pltpu.async_copy(ei.at[1, pl.ds(off + kof, CHUNK)],
                                 idx_d[b], dsem[b])

            def issue_scatter(b):
                pltpu.async_copy(rows[b], acc.at[idx_d[b]], ssem[b], add=True)

            def wait_gather(b):
                pltpu.make_async_copy(feats.at[idx_s.at[pl.ds(0, CHUNK)]],
                                      rows[b], gsem[b]).wait()
                pltpu.make_async_copy(ei.at[1, pl.ds(0, CHUNK)],
                                      idx_d[b], dsem[b]).wait()

            def wait_scatter(b):
                pltpu.make_async_copy(rows[b], acc.at[idx_d[b]], ssem[b]).wait()

            issue_gather(0, 0)

            def pair(g, carry):
                for b in (0, 1):
                    k = 2 * g + b
                    wait_gather(b)

                    @pl.when(k + 1 < cnt)
                    def _():
                        @pl.when(k >= 1)
                        def _():
                            wait_scatter(1 - b)

                        issue_gather(k + 1, 1 - b)

                    issue_scatter(b)
                return carry

            lax.fori_loop(0, cnt // 2, pair, 0)

            @pl.when(cnt % 2 == 1)
            def _():
                wait_gather(0)
                issue_scatter(0)

            wait_scatter(0)
            wait_scatter(1)

        @pl.when(c == 0)
        def _():
            run(0, c0, k0max)

        @pl.when(c != 0)
        def _():
            run(c0, c1, k1max)

        plsc.subcore_barrier()
        pltpu.sync_copy(acc.at[pl.ds(base, ROWS_PER_TILE)],
                        out.at[c, pl.ds(base, ROWS_PER_TILE)])

        @pl.when(s == N_SUB - 1)
        def _():
            pltpu.sync_copy(acc.at[pl.ds(TAIL_BASE, TAIL_ROWS)],
                            out.at[c, pl.ds(TAIL_BASE, TAIL_ROWS)])

    fn = pl.kernel(
        body,
        out_type=jax.ShapeDtypeStruct((N_CORES, N_NODES, D), jnp.float32),
        mesh=plsc.VectorSubcoreMesh(core_axis_name="c", subcore_axis_name="s",
                                    num_cores=N_CORES, num_subcores=N_SUB),
        scratch_types=[
            pltpu.VMEM((max(k0max, k1max) * CHUNK,), jnp.int32),
            pltpu.VMEM((CHUNK,), jnp.int32),
            pltpu.VMEM((CHUNK,), jnp.int32),
            pltpu.VMEM((CHUNK, D), jnp.float32),
            pltpu.VMEM((CHUNK, D), jnp.float32),
            pltpu.VMEM_SHARED((ACC_ROWS, D), jnp.float32),
            pltpu.SemaphoreType.DMA,
            pltpu.SemaphoreType.DMA,
            pltpu.SemaphoreType.DMA,
            pltpu.SemaphoreType.DMA,
            pltpu.SemaphoreType.DMA,
            pltpu.SemaphoreType.DMA,
            pltpu.SemaphoreType.DMA,
        ],
    )
    return fn(feats, ei, zeros)


ROW_BLK = 2000


def _mlp_body(p_ref, wa, ba, wb, bb, out_ref):
    h = p_ref[0] + p_ref[1]
    t = jnp.maximum(jnp.dot(h, wa[...], preferred_element_type=jnp.float32) + ba[...], 0.0)
    out_ref[...] = jnp.dot(t, wb[...], preferred_element_type=jnp.float32) + bb[...]


def _mlp(p, Wa, ba, Wb, bb):
    return pl.pallas_call(
        _mlp_body,
        grid=(N_NODES // ROW_BLK,),
        in_specs=[
            pl.BlockSpec((N_CORES, ROW_BLK, D), lambda i: (0, i, 0)),
            pl.BlockSpec((D, D), lambda i: (0, 0)),
            pl.BlockSpec((1, D), lambda i: (0, 0)),
            pl.BlockSpec((D, D), lambda i: (0, 0)),
            pl.BlockSpec((1, D), lambda i: (0, 0)),
        ],
        out_specs=pl.BlockSpec((ROW_BLK, D), lambda i: (i, 0)),
        out_shape=jax.ShapeDtypeStruct((N_NODES, D), jnp.float32),
    )(p, Wa, ba.reshape(1, D), Wb, bb.reshape(1, D))


def _mlp_pool_body(p_ref, wa, ba, wb, bb, wc, bcp, out_ref, acc):
    i = pl.program_id(0)

    @pl.when(i == 0)
    def _():
        acc[...] = jnp.zeros_like(acc)

    h = p_ref[0] + p_ref[1]
    t = jnp.maximum(jnp.dot(h, wa[...], preferred_element_type=jnp.float32) + ba[...], 0.0)
    h2 = jnp.dot(t, wb[...], preferred_element_type=jnp.float32) + bb[...]
    acc[...] += jnp.sum(h2, axis=0, keepdims=True)

    @pl.when(i == pl.num_programs(0) - 1)
    def _():
        out_ref[...] = jnp.dot(acc[...] * (1.0 / N_NODES), wc[...],
                               preferred_element_type=jnp.float32) + bcp[...]


def _mlp_pool(p, Wa, ba, Wb, bb, Wcp, bcp):
    return pl.pallas_call(
        _mlp_pool_body,
        grid=(N_NODES // ROW_BLK,),
        in_specs=[
            pl.BlockSpec((N_CORES, ROW_BLK, D), lambda i: (0, i, 0)),
            pl.BlockSpec((D, D), lambda i: (0, 0)),
            pl.BlockSpec((1, D), lambda i: (0, 0)),
            pl.BlockSpec((D, D), lambda i: (0, 0)),
            pl.BlockSpec((1, D), lambda i: (0, 0)),
            pl.BlockSpec((D, D), lambda i: (0, 0)),
            pl.BlockSpec((1, D), lambda i: (0, 0)),
        ],
        out_specs=pl.BlockSpec((1, D), lambda i: (0, 0)),
        out_shape=jax.ShapeDtypeStruct((1, D), jnp.float32),
        scratch_shapes=[pltpu.VMEM((1, D), jnp.float32)],
    )(p, Wa, ba.reshape(1, D), Wb, bb.reshape(1, D), Wcp, bcp)


def kernel(x, edge_index, W1a, b1a, W1b, b1b, W2a, b2a, W2b, b2b, Wc, bc):
    ei = edge_index.astype(jnp.int32)
    assert ei.shape[1] % CHUNK == 0
    zeros = jnp.zeros((CHUNK, D), jnp.float32)

    p1 = _sc_agg(x, ei, zeros)
    h1 = _mlp(p1, W1a, b1a, W1b, b1b)
    p2 = _sc_agg(h1, ei, zeros)

    n_cls = Wc.shape[1]
    Wcp = jnp.pad(Wc, ((0, 0), (0, D - n_cls)))
    bcp = jnp.pad(bc, (0, D - n_cls)).reshape(1, D)
    out = _mlp_pool(p2, W2a, b2a, W2b, b2b, Wcp, bcp)
    return out[:, :n_cls]


# speedup vs baseline: 1.0004x; 1.0004x over previous
import jax
import jax.numpy as jnp
from jax import lax
from jax.experimental import pallas as pl
from jax.experimental.pallas import tpu as pltpu
from jax.experimental.pallas import tpu_sc as plsc

N_NODES = 10000
D = 128
N_CORES = 2
N_SUB = 16
CHUNK = 128
SPLIT0 = 0.505
ROWS_PER_TILE = 624
TAIL_BASE = ROWS_PER_TILE * N_SUB
TAIL_ROWS = N_NODES - TAIL_BASE
ACC_ROWS = N_NODES


def _sc_agg(feats, ei, zeros):
    n_edges = ei.shape[1]
    t_chunks = n_edges // CHUNK
    c0 = int(round(t_chunks * SPLIT0))
    c1 = t_chunks - c0
    k0max = -(-c0 // N_SUB)
    k1max = -(-c1 // N_SUB)

    def body(feats, ei, zeros, out, idx_s, idx_d00, idx_d01, idx_d10, idx_d11,
             rows0, rows1, acc, gsem0, gsem1, ssem0, ssem1, dsem0, dsem1, stg):
        c = lax.axis_index("c")
        s = lax.axis_index("s")

        def cdims(cbase, csize):
            r_lo = s * csize // N_SUB
            cnt = (s + 1) * csize // N_SUB - r_lo
            off = pl.multiple_of((cbase + r_lo) * CHUNK, CHUNK)
            return off, cnt

        def stage(cbase, csize, kmax):
            off, _ = cdims(cbase, csize)
            pltpu.async_copy(ei.at[0, pl.ds(off, kmax * CHUNK)],
                             idx_s.at[pl.ds(0, kmax * CHUNK)], stg)

        @pl.when(c == 0)
        def _():
            stage(0, c0, k0max)

        @pl.when(c != 0)
        def _():
            stage(c0, c1, k1max)

        base = s * ROWS_PER_TILE

        @pl.when(c == 0)
        def _():
            pltpu.sync_copy(feats.at[pl.ds(base, ROWS_PER_TILE)],
                            acc.at[pl.ds(base, ROWS_PER_TILE)])

            @pl.when(s == N_SUB - 1)
            def _():
                pltpu.sync_copy(feats.at[pl.ds(TAIL_BASE, TAIL_ROWS)],
                                acc.at[pl.ds(TAIL_BASE, TAIL_ROWS)])

        @pl.when(c != 0)
        def _():
            for j in range(ROWS_PER_TILE // CHUNK):
                pltpu.sync_copy(zeros,
                                acc.at[pl.ds(base + j * CHUNK, CHUNK)])
            rem = ROWS_PER_TILE % CHUNK
            pltpu.sync_copy(zeros.at[pl.ds(0, rem)],
                            acc.at[pl.ds(base + ROWS_PER_TILE - rem, rem)])

            @pl.when(s == N_SUB - 1)
            def _():
                pltpu.sync_copy(zeros.at[pl.ds(0, TAIL_ROWS)],
                                acc.at[pl.ds(TAIL_BASE, TAIL_ROWS)])

        plsc.subcore_barrier()

        rows = (rows0, rows1)
        idx_d = ((idx_d00, idx_d01), (idx_d10, idx_d11))
        gsem = (gsem0, gsem1)
        ssem = (ssem0, ssem1)
        dsem = (dsem0, dsem1)

        def run(cbase, csize, kmax):
            off, cnt = cdims(cbase, csize)
            pltpu.make_async_copy(ei.at[0, pl.ds(0, kmax * CHUNK)],
                                  idx_s.at[pl.ds(0, kmax * CHUNK)], stg).wait()

            half = CHUNK // 2

            def issue_gather(k, b):
                kof = pl.multiple_of(k * CHUNK, CHUNK)
                for h in (0, 1):
                    hof = pl.multiple_of(kof + h * half, half)
                    pltpu.async_copy(feats.at[idx_s.at[pl.ds(hof, half)]],
                                     rows[b].at[pl.ds(h * half, half)], gsem[b])
                    pltpu.async_copy(ei.at[1, pl.ds(off + hof, half)],
                                     idx_d[b][h], dsem[b])

            def issue_scatter(b):
                for h in (0, 1):
                    pltpu.async_copy(rows[b].at[pl.ds(h * half, half)],
                                     acc.at[idx_d[b][h]], ssem[b], add=True)

            def wait_gather(b):
                for h in (0, 1):
                    pltpu.make_async_copy(feats.at[idx_s.at[pl.ds(0, half)]],
                                          rows[b].at[pl.ds(h * half, half)],
                                          gsem[b]).wait()
                    pltpu.make_async_copy(ei.at[1, pl.ds(0, half)],
                                          idx_d[b][h], dsem[b]).wait()

            def wait_scatter(b):
                for h in (0, 1):
                    pltpu.make_async_copy(rows[b].at[pl.ds(h * half, half)],
                                          acc.at[idx_d[b][h]], ssem[b]).wait()

            issue_gather(0, 0)

            def pair(g, carry):
                for b in (0, 1):
                    k = 2 * g + b
                    wait_gather(b)

                    @pl.when(k + 1 < cnt)
                    def _():
                        @pl.when(k >= 1)
                        def _():
                            wait_scatter(1 - b)

                        issue_gather(k + 1, 1 - b)

                    issue_scatter(b)
                return carry

            lax.fori_loop(0, cnt // 2, pair, 0)

            @pl.when(cnt % 2 == 1)
            def _():
                wait_gather(0)
                issue_scatter(0)

            wait_scatter(0)
            wait_scatter(1)

        @pl.when(c == 0)
        def _():
            run(0, c0, k0max)

        @pl.when(c != 0)
        def _():
            run(c0, c1, k1max)

        plsc.subcore_barrier()
        pltpu.sync_copy(acc.at[pl.ds(base, ROWS_PER_TILE)],
                        out.at[c, pl.ds(base, ROWS_PER_TILE)])

        @pl.when(s == N_SUB - 1)
        def _():
            pltpu.sync_copy(acc.at[pl.ds(TAIL_BASE, TAIL_ROWS)],
                            out.at[c, pl.ds(TAIL_BASE, TAIL_ROWS)])

    fn = pl.kernel(
        body,
        out_type=jax.ShapeDtypeStruct((N_CORES, N_NODES, D), jnp.float32),
        mesh=plsc.VectorSubcoreMesh(core_axis_name="c", subcore_axis_name="s",
                                    num_cores=N_CORES, num_subcores=N_SUB),
        scratch_types=[
            pltpu.VMEM((max(k0max, k1max) * CHUNK,), jnp.int32),
            pltpu.VMEM((CHUNK // 2,), jnp.int32),
            pltpu.VMEM((CHUNK // 2,), jnp.int32),
            pltpu.VMEM((CHUNK // 2,), jnp.int32),
            pltpu.VMEM((CHUNK // 2,), jnp.int32),
            pltpu.VMEM((CHUNK, D), jnp.float32),
            pltpu.VMEM((CHUNK, D), jnp.float32),
            pltpu.VMEM_SHARED((ACC_ROWS, D), jnp.float32),
            pltpu.SemaphoreType.DMA,
            pltpu.SemaphoreType.DMA,
            pltpu.SemaphoreType.DMA,
            pltpu.SemaphoreType.DMA,
            pltpu.SemaphoreType.DMA,
            pltpu.SemaphoreType.DMA,
            pltpu.SemaphoreType.DMA,
        ],
    )
    return fn(feats, ei, zeros)


ROW_BLK = 2000


def _mlp_body(p_ref, wa, ba, wb, bb, out_ref):
    h = p_ref[0] + p_ref[1]
    t = jnp.maximum(jnp.dot(h, wa[...], preferred_element_type=jnp.float32) + ba[...], 0.0)
    out_ref[...] = jnp.dot(t, wb[...], preferred_element_type=jnp.float32) + bb[...]


def _mlp(p, Wa, ba, Wb, bb):
    return pl.pallas_call(
        _mlp_body,
        grid=(N_NODES // ROW_BLK,),
        in_specs=[
            pl.BlockSpec((N_CORES, ROW_BLK, D), lambda i: (0, i, 0)),
            pl.BlockSpec((D, D), lambda i: (0, 0)),
            pl.BlockSpec((1, D), lambda i: (0, 0)),
            pl.BlockSpec((D, D), lambda i: (0, 0)),
            pl.BlockSpec((1, D), lambda i: (0, 0)),
        ],
        out_specs=pl.BlockSpec((ROW_BLK, D), lambda i: (i, 0)),
        out_shape=jax.ShapeDtypeStruct((N_NODES, D), jnp.float32),
    )(p, Wa, ba.reshape(1, D), Wb, bb.reshape(1, D))


def _mlp_pool_body(p_ref, wa, ba, wb, bb, wc, bcp, out_ref, acc):
    i = pl.program_id(0)

    @pl.when(i == 0)
    def _():
        acc[...] = jnp.zeros_like(acc)

    h = p_ref[0] + p_ref[1]
    t = jnp.maximum(jnp.dot(h, wa[...], preferred_element_type=jnp.float32) + ba[...], 0.0)
    h2 = jnp.dot(t, wb[...], preferred_element_type=jnp.float32) + bb[...]
    acc[...] += jnp.sum(h2, axis=0, keepdims=True)

    @pl.when(i == pl.num_programs(0) - 1)
    def _():
        out_ref[...] = jnp.dot(acc[...] * (1.0 / N_NODES), wc[...],
                               preferred_element_type=jnp.float32) + bcp[...]


def _mlp_pool(p, Wa, ba, Wb, bb, Wcp, bcp):
    return pl.pallas_call(
        _mlp_pool_body,
        grid=(N_NODES // ROW_BLK,),
        in_specs=[
            pl.BlockSpec((N_CORES, ROW_BLK, D), lambda i: (0, i, 0)),
            pl.BlockSpec((D, D), lambda i: (0, 0)),
            pl.BlockSpec((1, D), lambda i: (0, 0)),
            pl.BlockSpec((D, D), lambda i: (0, 0)),
            pl.BlockSpec((1, D), lambda i: (0, 0)),
            pl.BlockSpec((D, D), lambda i: (0, 0)),
            pl.BlockSpec((1, D), lambda i: (0, 0)),
        ],
        out_specs=pl.BlockSpec((1, D), lambda i: (0, 0)),
        out_shape=jax.ShapeDtypeStruct((1, D), jnp.float32),
        scratch_shapes=[pltpu.VMEM((1, D), jnp.float32)],
    )(p, Wa, ba.reshape(1, D), Wb, bb.reshape(1, D), Wcp, bcp)


def kernel(x, edge_index, W1a, b1a, W1b, b1b, W2a, b2a, W2b, b2b, Wc, bc):
    ei = edge_index.astype(jnp.int32)
    assert ei.shape[1] % CHUNK == 0
    zeros = jnp.zeros((CHUNK, D), jnp.float32)

    p1 = _sc_agg(x, ei, zeros)
    h1 = _mlp(p1, W1a, b1a, W1b, b1b)
    p2 = _sc_agg(h1, ei, zeros)

    n_cls = Wc.shape[1]
    Wcp = jnp.pad(Wc, ((0, 0), (0, D - n_cls)))
    bcp = jnp.pad(bc, (0, D - n_cls)).reshape(1, D)
    out = _mlp_pool(p2, W2a, b2a, W2b, b2b, Wcp, bcp)
    return out[:, :n_cls]


# speedup vs baseline: 1.0041x; 1.0038x over previous
import jax
import jax.numpy as jnp
from jax import lax
from jax.experimental import pallas as pl
from jax.experimental.pallas import tpu as pltpu
from jax.experimental.pallas import tpu_sc as plsc

N_NODES = 10000
D = 128
N_CORES = 2
N_SUB = 16
CHUNK = 128
SPLIT0 = 0.505
ROWS_PER_TILE = 624
TAIL_BASE = ROWS_PER_TILE * N_SUB
TAIL_ROWS = N_NODES - TAIL_BASE
ACC_ROWS = N_NODES


def _sc_agg(feats, ei, zeros):
    n_edges = ei.shape[1]
    t_chunks = n_edges // CHUNK
    c0 = int(round(t_chunks * SPLIT0))
    c1 = t_chunks - c0
    k0max = -(-c0 // N_SUB)
    k1max = -(-c1 // N_SUB)

    def body(feats, ei, zeros, out, idx_s, idx_d0, idx_d1, rows0, rows1, acc,
             gsem0, gsem1, ssem0, ssem1, dsem0, dsem1, stg):
        c = lax.axis_index("c")
        s = lax.axis_index("s")

        def cdims(cbase, csize):
            r_lo = s * csize // N_SUB
            cnt = (s + 1) * csize // N_SUB - r_lo
            off = pl.multiple_of((cbase + r_lo) * CHUNK, CHUNK)
            return off, cnt

        def stage(cbase, csize, kmax):
            off, _ = cdims(cbase, csize)
            pltpu.async_copy(ei.at[0, pl.ds(off, kmax * CHUNK)],
                             idx_s.at[pl.ds(0, kmax * CHUNK)], stg)

        @pl.when(c == 0)
        def _():
            stage(0, c0, k0max)

        @pl.when(c != 0)
        def _():
            stage(c0, c1, k1max)

        base = s * ROWS_PER_TILE

        @pl.when(c == 0)
        def _():
            pltpu.sync_copy(feats.at[pl.ds(base, ROWS_PER_TILE)],
                            acc.at[pl.ds(base, ROWS_PER_TILE)])

            @pl.when(s == N_SUB - 1)
            def _():
                pltpu.sync_copy(feats.at[pl.ds(TAIL_BASE, TAIL_ROWS)],
                                acc.at[pl.ds(TAIL_BASE, TAIL_ROWS)])

        @pl.when(c != 0)
        def _():
            for j in range(ROWS_PER_TILE // CHUNK):
                pltpu.sync_copy(zeros,
                                acc.at[pl.ds(base + j * CHUNK, CHUNK)])
            rem = ROWS_PER_TILE % CHUNK
            pltpu.sync_copy(zeros.at[pl.ds(0, rem)],
                            acc.at[pl.ds(base + ROWS_PER_TILE - rem, rem)])

            @pl.when(s == N_SUB - 1)
            def _():
                pltpu.sync_copy(zeros.at[pl.ds(0, TAIL_ROWS)],
                                acc.at[pl.ds(TAIL_BASE, TAIL_ROWS)])

        plsc.subcore_barrier()

        rows = (rows0, rows1)
        idx_d = (idx_d0, idx_d1)
        gsem = (gsem0, gsem1)
        ssem = (ssem0, ssem1)
        dsem = (dsem0, dsem1)

        def run(cbase, csize, kmax):
            off, cnt = cdims(cbase, csize)
            pltpu.make_async_copy(ei.at[0, pl.ds(0, kmax * CHUNK)],
                                  idx_s.at[pl.ds(0, kmax * CHUNK)], stg).wait()

            def issue_gather(k, b):
                kof = pl.multiple_of(k * CHUNK, CHUNK)
                pltpu.async_copy(feats.at[idx_s.at[pl.ds(kof, CHUNK)]],
                                 rows[b], gsem[b])
                pltpu.async_copy(ei.at[1, pl.ds(off + kof, CHUNK)],
                                 idx_d[b], dsem[b])

            def issue_scatter(b):
                pltpu.async_copy(rows[b], acc.at[idx_d[b]], ssem[b], add=True)

            def wait_gather(b):
                pltpu.make_async_copy(feats.at[idx_s.at[pl.ds(0, CHUNK)]],
                                      rows[b], gsem[b]).wait()
                pltpu.make_async_copy(ei.at[1, pl.ds(0, CHUNK)],
                                      idx_d[b], dsem[b]).wait()

            def wait_scatter(b):
                pltpu.make_async_copy(rows[b], acc.at[idx_d[b]], ssem[b]).wait()

            issue_gather(0, 0)

            def pair(g, carry):
                for b in (0, 1):
                    k = 2 * g + b
                    wait_gather(b)

                    @pl.when(k + 1 < cnt)
                    def _():
                        @pl.when(k >= 1)
                        def _():
                            wait_scatter(1 - b)

                        issue_gather(k + 1, 1 - b)

                    issue_scatter(b)
                return carry

            lax.fori_loop(0, cnt // 2, pair, 0)

            @pl.when(cnt % 2 == 1)
            def _():
                wait_gather(0)
                issue_scatter(0)

            wait_scatter(0)
            wait_scatter(1)

        @pl.when(c == 0)
        def _():
            run(0, c0, k0max)

        @pl.when(c != 0)
        def _():
            run(c0, c1, k1max)

        plsc.subcore_barrier()
        pltpu.sync_copy(acc.at[pl.ds(base, ROWS_PER_TILE)],
                        out.at[c, pl.ds(base, ROWS_PER_TILE)])

        @pl.when(s == N_SUB - 1)
        def _():
            pltpu.sync_copy(acc.at[pl.ds(TAIL_BASE, TAIL_ROWS)],
                            out.at[c, pl.ds(TAIL_BASE, TAIL_ROWS)])

    fn = pl.kernel(
        body,
        out_type=jax.ShapeDtypeStruct((N_CORES, N_NODES, D), jnp.float32),
        mesh=plsc.VectorSubcoreMesh(core_axis_name="c", subcore_axis_name="s",
                                    num_cores=N_CORES, num_subcores=N_SUB),
        scratch_types=[
            pltpu.VMEM((max(k0max, k1max) * CHUNK,), jnp.int32),
            pltpu.VMEM((CHUNK,), jnp.int32),
            pltpu.VMEM((CHUNK,), jnp.int32),
            pltpu.VMEM((CHUNK, D), jnp.float32),
            pltpu.VMEM((CHUNK, D), jnp.float32),
            pltpu.VMEM_SHARED((ACC_ROWS, D), jnp.float32),
            pltpu.SemaphoreType.DMA,
            pltpu.SemaphoreType.DMA,
            pltpu.SemaphoreType.DMA,
            pltpu.SemaphoreType.DMA,
            pltpu.SemaphoreType.DMA,
            pltpu.SemaphoreType.DMA,
            pltpu.SemaphoreType.DMA,
        ],
    )
    return fn(feats, ei, zeros)


ROW_BLK = 2000


def _mlp_body(p_ref, wa, ba, wb, bb, out_ref):
    h = p_ref[0] + p_ref[1]
    t = jnp.maximum(jnp.dot(h, wa[...], preferred_element_type=jnp.float32) + ba[...], 0.0)
    out_ref[...] = jnp.dot(t, wb[...], preferred_element_type=jnp.float32) + bb[...]


def _mlp(p, Wa, ba, Wb, bb):
    return pl.pallas_call(
        _mlp_body,
        grid=(N_NODES // ROW_BLK,),
        in_specs=[
            pl.BlockSpec((N_CORES, ROW_BLK, D), lambda i: (0, i, 0)),
            pl.BlockSpec((D, D), lambda i: (0, 0)),
            pl.BlockSpec((1, D), lambda i: (0, 0)),
            pl.BlockSpec((D, D), lambda i: (0, 0)),
            pl.BlockSpec((1, D), lambda i: (0, 0)),
        ],
        out_specs=pl.BlockSpec((ROW_BLK, D), lambda i: (i, 0)),
        out_shape=jax.ShapeDtypeStruct((N_NODES, D), jnp.float32),
    )(p, Wa, ba.reshape(1, D), Wb, bb.reshape(1, D))


def _mlp_pool_body(p_ref, wa, ba, wb, bb, wc, bcp, out_ref, acc):
    i = pl.program_id(0)

    @pl.when(i == 0)
    def _():
        acc[...] = jnp.zeros_like(acc)

    h = p_ref[0] + p_ref[1]
    t = jnp.maximum(jnp.dot(h, wa[...], preferred_element_type=jnp.float32) + ba[...], 0.0)
    h2 = jnp.dot(t, wb[...], preferred_element_type=jnp.float32) + bb[...]
    acc[...] += jnp.sum(h2, axis=0, keepdims=True)

    @pl.when(i == pl.num_programs(0) - 1)
    def _():
        out_ref[...] = jnp.dot(acc[...] * (1.0 / N_NODES), wc[...],
                               preferred_element_type=jnp.float32) + bcp[...]


def _mlp_pool(p, Wa, ba, Wb, bb, Wcp, bcp):
    return pl.pallas_call(
        _mlp_pool_body,
        grid=(N_NODES // ROW_BLK,),
        in_specs=[
            pl.BlockSpec((N_CORES, ROW_BLK, D), lambda i: (0, i, 0)),
            pl.BlockSpec((D, D), lambda i: (0, 0)),
            pl.BlockSpec((1, D), lambda i: (0, 0)),
            pl.BlockSpec((D, D), lambda i: (0, 0)),
            pl.BlockSpec((1, D), lambda i: (0, 0)),
            pl.BlockSpec((D, D), lambda i: (0, 0)),
            pl.BlockSpec((1, D), lambda i: (0, 0)),
        ],
        out_specs=pl.BlockSpec((1, D), lambda i: (0, 0)),
        out_shape=jax.ShapeDtypeStruct((1, D), jnp.float32),
        scratch_shapes=[pltpu.VMEM((1, D), jnp.float32)],
    )(p, Wa, ba.reshape(1, D), Wb, bb.reshape(1, D), Wcp, bcp)


def kernel(x, edge_index, W1a, b1a, W1b, b1b, W2a, b2a, W2b, b2b, Wc, bc):
    ei = edge_index.astype(jnp.int32)
    assert ei.shape[1] % CHUNK == 0
    zeros = jnp.zeros((CHUNK, D), jnp.float32)

    p1 = _sc_agg(x, ei, zeros)
    h1 = _mlp(p1, W1a, b1a, W1b, b1b)
    p2 = _sc_agg(h1, ei, zeros)

    n_cls = Wc.shape[1]
    Wcp = jnp.pad(Wc, ((0, 0), (0, D - n_cls)))
    bcp = jnp.pad(bc, (0, D - n_cls)).reshape(1, D)
    out = _mlp_pool(p2, W2a, b2a, W2b, b2b, Wcp, bcp)
    return out[:, :n_cls]
